# Initial kernel scaffold; baseline (speedup 1.0000x reference)
import jax
import jax.numpy as jnp
from jax.experimental import pallas as pl


def kernel(input_tensor, pos_table, W, b, gamma, beta):
    raise NotImplementedError("write your pallas kernel here")


# speedup vs baseline: 2.5953x; 2.5953x over previous
import functools

import jax
import jax.numpy as jnp
from jax.experimental import pallas as pl
from jax.experimental.pallas import tpu as pltpu

EPS = 1e-12


def _body(x_ref, p_ref, w_ref, b_ref, g_ref, be_ref, o_ref):
    x = x_ref[0] + p_ref[...]
    y = jnp.dot(x, w_ref[...], preferred_element_type=jnp.float32)
    y = y + b_ref[...]
    u = jnp.mean(y, axis=-1, keepdims=True)
    s = jnp.mean((y - u) ** 2, axis=-1, keepdims=True)
    yn = (y - u) * jax.lax.rsqrt(s + EPS)
    o_ref[0] = g_ref[...] * yn + be_ref[...]


@functools.partial(jax.jit, static_argnames=())
def kernel(input_tensor, pos_table, W, b, gamma, beta):
    B, S, D = input_tensor.shape
    DH = W.shape[1]
    BM = min(512, S)
    n_s = S // BM

    grid = (n_s, B)
    out = pl.pallas_call(
        _body,
        grid=grid,
        in_specs=[
            pl.BlockSpec((1, BM, D), lambda s, bi: (bi, s, 0)),
            pl.BlockSpec((BM, D), lambda s, bi: (s, 0)),
            pl.BlockSpec((D, DH), lambda s, bi: (0, 0)),
            pl.BlockSpec((1, DH), lambda s, bi: (0, 0)),
            pl.BlockSpec((1, DH), lambda s, bi: (0, 0)),
            pl.BlockSpec((1, DH), lambda s, bi: (0, 0)),
        ],
        out_specs=pl.BlockSpec((1, BM, DH), lambda s, bi: (bi, s, 0)),
        out_shape=jax.ShapeDtypeStruct((B, S, DH), jnp.float32),
        compiler_params=pltpu.CompilerParams(
            dimension_semantics=("arbitrary", "arbitrary"),
        ),
    )(
        input_tensor,
        pos_table,
        W,
        b.reshape(1, DH),
        gamma.reshape(1, DH),
        beta.reshape(1, DH),
    )
    return out


# speedup vs baseline: 2.6042x; 1.0034x over previous
import functools

import jax
import jax.numpy as jnp
from jax.experimental import pallas as pl
from jax.experimental.pallas import tpu as pltpu

EPS = 1e-12


def _body(x_ref, p_ref, w_ref, b_ref, g_ref, be_ref, o_ref):
    x = (x_ref[0] + p_ref[...]).astype(jnp.bfloat16)
    y = jnp.dot(x, w_ref[...].astype(jnp.bfloat16),
                preferred_element_type=jnp.float32)
    y = y + b_ref[...]
    u = jnp.mean(y, axis=-1, keepdims=True)
    s = jnp.mean((y - u) ** 2, axis=-1, keepdims=True)
    yn = (y - u) * jax.lax.rsqrt(s + EPS)
    o_ref[0] = g_ref[...] * yn + be_ref[...]


@functools.partial(jax.jit, static_argnames=())
def kernel(input_tensor, pos_table, W, b, gamma, beta):
    B, S, D = input_tensor.shape
    DH = W.shape[1]
    BM = min(512, S)
    n_s = S // BM

    grid = (n_s, B)
    out = pl.pallas_call(
        _body,
        grid=grid,
        in_specs=[
            pl.BlockSpec((1, BM, D), lambda s, bi: (bi, s, 0)),
            pl.BlockSpec((BM, D), lambda s, bi: (s, 0)),
            pl.BlockSpec((D, DH), lambda s, bi: (0, 0)),
            pl.BlockSpec((1, DH), lambda s, bi: (0, 0)),
            pl.BlockSpec((1, DH), lambda s, bi: (0, 0)),
            pl.BlockSpec((1, DH), lambda s, bi: (0, 0)),
        ],
        out_specs=pl.BlockSpec((1, BM, DH), lambda s, bi: (bi, s, 0)),
        out_shape=jax.ShapeDtypeStruct((B, S, DH), jnp.float32),
        compiler_params=pltpu.CompilerParams(
            dimension_semantics=("arbitrary", "arbitrary"),
        ),
    )(
        input_tensor,
        pos_table,
        W,
        b.reshape(1, DH),
        gamma.reshape(1, DH),
        beta.reshape(1, DH),
    )
    return out


# speedup vs baseline: 2.9967x; 1.1508x over previous
import functools

import jax
import jax.numpy as jnp
from jax.experimental import pallas as pl
from jax.experimental.pallas import tpu as pltpu

EPS = 1e-12


def _body(x_ref, p_ref, w_ref, b_ref, g_ref, be_ref, o_ref):
    x = x_ref[0] + p_ref[...]
    y = jnp.dot(x, w_ref[...], preferred_element_type=jnp.float32)
    y = y + b_ref[...]
    u = jnp.mean(y, axis=-1, keepdims=True)
    s = jnp.mean((y - u) ** 2, axis=-1, keepdims=True)
    yn = (y - u) * jax.lax.rsqrt(s + EPS)
    o_ref[0] = g_ref[...] * yn + be_ref[...]


@functools.partial(jax.jit, static_argnames=())
def kernel(input_tensor, pos_table, W, b, gamma, beta):
    B, S, D = input_tensor.shape
    DH = W.shape[1]
    BM = min(1024, S)
    n_s = S // BM

    grid = (n_s, B)
    out = pl.pallas_call(
        _body,
        grid=grid,
        in_specs=[
            pl.BlockSpec((1, BM, D), lambda s, bi: (bi, s, 0)),
            pl.BlockSpec((BM, D), lambda s, bi: (s, 0)),
            pl.BlockSpec((D, DH), lambda s, bi: (0, 0)),
            pl.BlockSpec((1, DH), lambda s, bi: (0, 0)),
            pl.BlockSpec((1, DH), lambda s, bi: (0, 0)),
            pl.BlockSpec((1, DH), lambda s, bi: (0, 0)),
        ],
        out_specs=pl.BlockSpec((1, BM, DH), lambda s, bi: (bi, s, 0)),
        out_shape=jax.ShapeDtypeStruct((B, S, DH), jnp.float32),
        compiler_params=pltpu.CompilerParams(
            dimension_semantics=("arbitrary", "arbitrary"),
        ),
    )(
        input_tensor,
        pos_table,
        W,
        b.reshape(1, DH),
        gamma.reshape(1, DH),
        beta.reshape(1, DH),
    )
    return out


# speedup vs baseline: 3.0060x; 1.0031x over previous
import functools

import jax
import jax.numpy as jnp
from jax.experimental import pallas as pl
from jax.experimental.pallas import tpu as pltpu

EPS = 1e-12


def _body(x_ref, p_ref, w_ref, b_ref, g_ref, be_ref, o_ref):
    x = x_ref[0] + p_ref[...]
    y = jnp.dot(x, w_ref[...], preferred_element_type=jnp.float32)
    y = y + b_ref[...]
    u = jnp.mean(y, axis=-1, keepdims=True)
    s = jnp.mean((y - u) ** 2, axis=-1, keepdims=True)
    yn = (y - u) * jax.lax.rsqrt(s + EPS)
    o_ref[0] = g_ref[...] * yn + be_ref[...]


@functools.partial(jax.jit, static_argnames=())
def kernel(input_tensor, pos_table, W, b, gamma, beta):
    B, S, D = input_tensor.shape
    DH = W.shape[1]
    BM = min(1024, S)
    n_s = S // BM

    grid = (n_s, B)
    out = pl.pallas_call(
        _body,
        grid=grid,
        in_specs=[
            pl.BlockSpec((1, BM, D), lambda s, bi: (bi, s, 0)),
            pl.BlockSpec((BM, D), lambda s, bi: (s, 0)),
            pl.BlockSpec((D, DH), lambda s, bi: (0, 0)),
            pl.BlockSpec((1, DH), lambda s, bi: (0, 0)),
            pl.BlockSpec((1, DH), lambda s, bi: (0, 0)),
            pl.BlockSpec((1, DH), lambda s, bi: (0, 0)),
        ],
        out_specs=pl.BlockSpec((1, BM, DH), lambda s, bi: (bi, s, 0)),
        out_shape=jax.ShapeDtypeStruct((B, S, DH), jnp.float32),
        compiler_params=pltpu.CompilerParams(
            dimension_semantics=("parallel", "parallel"),
        ),
    )(
        input_tensor,
        pos_table,
        W,
        b.reshape(1, DH),
        gamma.reshape(1, DH),
        beta.reshape(1, DH),
    )
    return out


# speedup vs baseline: 3.0302x; 1.0081x over previous
import functools

import jax
import jax.numpy as jnp
from jax.experimental import pallas as pl
from jax.experimental.pallas import tpu as pltpu

EPS = 1e-12


def _body(x_ref, p_ref, w_ref, b_ref, g_ref, be_ref, o_ref):
    x = x_ref[0] + p_ref[...]
    y = jnp.dot(x, w_ref[...], preferred_element_type=jnp.float32)
    t = y + b_ref[...]
    m1 = jnp.mean(t, axis=-1, keepdims=True)
    m2 = jnp.mean(t * t, axis=-1, keepdims=True)
    r = jax.lax.rsqrt(m2 - m1 * m1 + EPS)
    o_ref[0] = (t - m1) * r * g_ref[...] + be_ref[...]


@functools.partial(jax.jit, static_argnames=())
def kernel(input_tensor, pos_table, W, b, gamma, beta):
    B, S, D = input_tensor.shape
    DH = W.shape[1]
    BM = min(1024, S)
    n_s = S // BM

    grid = (n_s, B)
    out = pl.pallas_call(
        _body,
        grid=grid,
        in_specs=[
            pl.BlockSpec((1, BM, D), lambda s, bi: (bi, s, 0)),
            pl.BlockSpec((BM, D), lambda s, bi: (s, 0)),
            pl.BlockSpec((D, DH), lambda s, bi: (0, 0)),
            pl.BlockSpec((1, DH), lambda s, bi: (0, 0)),
            pl.BlockSpec((1, DH), lambda s, bi: (0, 0)),
            pl.BlockSpec((1, DH), lambda s, bi: (0, 0)),
        ],
        out_specs=pl.BlockSpec((1, BM, DH), lambda s, bi: (bi, s, 0)),
        out_shape=jax.ShapeDtypeStruct((B, S, DH), jnp.float32),
        compiler_params=pltpu.CompilerParams(
            dimension_semantics=("parallel", "parallel"),
        ),
    )(
        input_tensor,
        pos_table,
        W,
        b.reshape(1, DH),
        gamma.reshape(1, DH),
        beta.reshape(1, DH),
    )
    return out
